# Initial kernel scaffold; baseline (speedup 1.0000x reference)
import jax
import jax.numpy as jnp
from jax.experimental import pallas as pl


def kernel(x, edge_index, batch, node_num, edge_num, start_node, gid, checkStatus, W1, b1, g1, be1, W2, b2, g2, be2, W3, b3, g3, be3, lW1, lb1, lW2, lb2):
    raise NotImplementedError("write your pallas kernel here")


# speedup vs baseline: 6.0350x; 6.0350x over previous
import functools
import jax
import jax.numpy as jnp
from jax import lax
from jax.experimental import pallas as pl
from jax.experimental.pallas import tpu as pltpu
from jax.experimental.pallas import tpu_sc as plsc

_N = 50000
_E = 800000
_G = 64
_EP = 802816
_ROWS_PT = 3128
_NPAD = 16 * _ROWS_PT
_TR = 1000
_RPT = 392


def _make_segsum(nb, F, rounds, ch, nchunk, ct):
  assert nchunk * ch + ct == _RPT
  mesh = plsc.VectorSubcoreMesh(core_axis_name="c", subcore_axis_name="s")

  @functools.partial(
      pl.kernel, mesh=mesh,
      compiler_params=pltpu.CompilerParams(use_tc_tiling_on_sc=False),
      out_type=jax.ShapeDtypeStruct((nb, _NPAD, F), jnp.float32),
      scratch_types=[
          pltpu.VMEM((ch, 128), jnp.int32),
          pltpu.VMEM((ch, 128), jnp.int32),
          pltpu.VMEM((ch, 128, F), jnp.float32),
          pltpu.VMEM_SHARED((_NPAD, F), jnp.float32),
          pltpu.SemaphoreType.DMA,
      ],
  )
  def k(src_hbm, dst_hbm, val_hbm, zrows_hbm, out_hbm, src_v, dst_v, rows_v,
        acc, sem):
    c = lax.axis_index("c")
    s = lax.axis_index("s")
    row0 = s * _RPT

    for r in range(rounds):
      b = 2 * r + c
      pltpu.sync_copy(zrows_hbm, acc.at[pl.ds(s * _ROWS_PT, _ROWS_PT)])
      plsc.subcore_barrier()

      def gather_scatter(base_row, nrow):
        pltpu.sync_copy(src_hbm.at[pl.ds(base_row, nrow)],
                        src_v.at[pl.ds(0, nrow)])
        pltpu.sync_copy(dst_hbm.at[pl.ds(base_row, nrow)],
                        dst_v.at[pl.ds(0, nrow)])
        cps = []
        for j in range(nrow):
          cps.append(pltpu.async_copy(
              val_hbm.at[b].at[src_v.at[j]], rows_v.at[j], sem))
        for cp in cps:
          cp.wait()
        for j in range(nrow):
          pltpu.sync_copy(rows_v.at[j], acc.at[dst_v.at[j]], add=True)

      def body(kk, carry):
        gather_scatter(row0 + kk * ch, ch)
        return carry

      lax.fori_loop(0, nchunk, body, 0)
      if ct:
        gather_scatter(row0 + nchunk * ch, ct)

      plsc.subcore_barrier()
      pltpu.sync_copy(acc.at[pl.ds(s * _ROWS_PT, _ROWS_PT)],
                      out_hbm.at[b].at[pl.ds(s * _ROWS_PT, _ROWS_PT)])
      plsc.subcore_barrier()

  return k


_segsum_l1 = _make_segsum(2, 32, 1, 4, 98, 0)
_segsum_l2 = _make_segsum(4, 32, 2, 4, 98, 0)
_segsum_l3 = _make_segsum(2, 16, 1, 16, 24, 8)


def _unblock(a):
  nb, tr, f = a.shape
  return jnp.transpose(a, (1, 0, 2)).reshape(tr, nb * f)


def _block(a, nb):
  tr, d = a.shape
  return jnp.transpose(a.reshape(tr, nb, d // nb), (1, 0, 2))


def _acc_stats(i, st_ref, y):
  ps = jnp.concatenate([jnp.sum(y, 0, keepdims=True),
                        jnp.sum(y * y, 0, keepdims=True)], axis=0)

  @pl.when(i == 0)
  def _():
    st_ref[...] = jnp.zeros_like(st_ref)

  st_ref[...] += ps


def _bn_from_stats(st, g, be, y):
  m = st[0:1, :] / _N
  var = st[1:2, :] / _N - m * m
  inv = 1.0 / jnp.sqrt(var + 1e-5)
  return (y - m) * inv * g + be


def _tca_body(x_ref, aggb_ref, w_ref, b_ref, y_ref, st_ref):
  i = pl.program_id(0)
  sfeat = x_ref[...] + _unblock(aggb_ref[...])
  y = jnp.dot(sfeat, w_ref[...], preferred_element_type=jnp.float32) + b_ref[...]
  y_ref[...] = y
  _acc_stats(i, st_ref, y)


def _tcb_body(y_ref, st_ref, g_ref, be_ref, w_ref, out_ref, nb):
  h = _bn_from_stats(st_ref[...], g_ref[...], be_ref[...], y_ref[...])
  h = jnp.maximum(h, 0.0)
  y2 = jnp.dot(h, w_ref[...], preferred_element_type=jnp.float32)
  out_ref[...] = _block(y2, nb)


def _tcc_body(yb_ref, aggb_ref, b_ref, h_ref, st_ref):
  i = pl.program_id(0)
  t = _unblock(yb_ref[...]) + _unblock(aggb_ref[...]) + b_ref[...]
  h_ref[...] = t
  _acc_stats(i, st_ref, t)


def _tcf_body(h_ref, st_ref, g_ref, be_ref, w1_ref, c1_ref, w2_ref, c2_ref,
              batch_ref, sums_ref, cnt_ref, mean_ref):
  i = pl.program_id(0)
  h3 = _bn_from_stats(st_ref[...], g_ref[...], be_ref[...], h_ref[...])
  z = jnp.maximum(
      jnp.dot(h3, w1_ref[...], preferred_element_type=jnp.float32)
      + c1_ref[...], 0.0)
  z2 = jnp.dot(z, w2_ref[...], preferred_element_type=jnp.float32) + c2_ref[...]
  p = 1.0 / (1.0 + jnp.exp(-z2))
  gidx = lax.broadcasted_iota(jnp.int32, (1, _G), 1).astype(jnp.float32)
  oh = (batch_ref[...] == gidx).astype(jnp.float32)
  dn = (((0,), (0,)), ((), ()))
  psum = lax.dot_general(oh, p, dn, preferred_element_type=jnp.float32)
  csum = lax.dot_general(oh, jnp.ones_like(p), dn,
                         preferred_element_type=jnp.float32)

  @pl.when(i == 0)
  def _():
    sums_ref[...] = jnp.zeros_like(sums_ref)
    cnt_ref[...] = jnp.zeros_like(cnt_ref)

  sums_ref[...] += psum
  cnt_ref[...] += csum

  @pl.when(i == pl.num_programs(0) - 1)
  def _():
    mean_ref[...] = sums_ref[...] / jnp.maximum(cnt_ref[...], 1.0)


_GRID = _N // _TR


def _rowspec(d):
  return pl.BlockSpec((_TR, d), lambda i: (i, 0))


def _blkspec(nb, f):
  return pl.BlockSpec((nb, _TR, f), lambda i: (0, i, 0))


def _fullspec(shape):
  nd = len(shape)
  return pl.BlockSpec(shape, lambda i, _n=nd: (0,) * _n)


def _tc_a(x, aggb, w1t, b1):
  return pl.pallas_call(
      _tca_body,
      grid=(_GRID,),
      in_specs=[_rowspec(64), _blkspec(2, 32), _fullspec((64, 256)),
                _fullspec((1, 256))],
      out_specs=[_rowspec(256), _fullspec((2, 256))],
      out_shape=[jax.ShapeDtypeStruct((_N, 256), jnp.float32),
                 jax.ShapeDtypeStruct((2, 256), jnp.float32)],
  )(x, aggb, w1t, b1)


def _tc_b(y1, st1, g1, be1, w2t, din, dout, nb):
  body = functools.partial(_tcb_body, nb=nb)
  return pl.pallas_call(
      body,
      grid=(_GRID,),
      in_specs=[_rowspec(din), _fullspec((2, din)), _fullspec((1, din)),
                _fullspec((1, din)), _fullspec((din, dout))],
      out_specs=[_blkspec(nb, dout // nb)],
      out_shape=[jax.ShapeDtypeStruct((nb, _NPAD, dout // nb), jnp.float32)],
  )(y1, st1, g1, be1, w2t)[0]


def _tc_c(yb, aggb, b, nb, f):
  d = nb * f
  return pl.pallas_call(
      _tcc_body,
      grid=(_GRID,),
      in_specs=[_blkspec(nb, f), _blkspec(nb, f), _fullspec((1, d))],
      out_specs=[_rowspec(d), _fullspec((2, d))],
      out_shape=[jax.ShapeDtypeStruct((_N, d), jnp.float32),
                 jax.ShapeDtypeStruct((2, d), jnp.float32)],
  )(yb, aggb, b)


def _tc_f(h3pre, st3, g3, be3, lw1t, lb1, lw2t, lb2, batchf):
  outs = pl.pallas_call(
      _tcf_body,
      grid=(_GRID,),
      in_specs=[_rowspec(32), _fullspec((2, 32)), _fullspec((1, 32)),
                _fullspec((1, 32)), _fullspec((32, 32)), _fullspec((1, 32)),
                _fullspec((32, 1)), _fullspec((1, 1)), _rowspec(1)],
      out_specs=[_fullspec((_G, 1)), _fullspec((_G, 1)), _fullspec((_G, 1))],
      out_shape=[jax.ShapeDtypeStruct((_G, 1), jnp.float32),
                 jax.ShapeDtypeStruct((_G, 1), jnp.float32),
                 jax.ShapeDtypeStruct((_G, 1), jnp.float32)],
  )(h3pre, st3, g3, be3, lw1t, lb1, lw2t, lb2, batchf)
  return outs[2]


def kernel(x, edge_index, batch, node_num, edge_num, start_node, gid,
           checkStatus, W1, b1, g1, be1, W2, b2, g2, be2, W3, b3, g3, be3,
           lW1, lb1, lW2, lb2):
  pad = _EP - _E
  src = jnp.concatenate([edge_index[0], jnp.zeros((pad,), jnp.int32)])
  dst = jnp.concatenate([edge_index[1], jnp.full((pad,), _N, jnp.int32)])
  src2 = src.reshape(_EP // 128, 128)
  dst2 = dst.reshape(_EP // 128, 128)
  z32 = jnp.zeros((_ROWS_PT, 32), jnp.float32)
  z16 = jnp.zeros((_ROWS_PT, 16), jnp.float32)

  xb = jnp.transpose(x.reshape(_N, 2, 32), (1, 0, 2))
  agg1b = _segsum_l1(src2, dst2, xb, z32)
  y1, st1 = _tc_a(x, agg1b, W1.T, b1[None, :])

  y2b = _tc_b(y1, st1, g1[None, :], be1[None, :], W2.T, 256, 128, 4)
  agg2b = _segsum_l2(src2, dst2, y2b, z32)
  h2pre, st2 = _tc_c(y2b, agg2b, b2[None, :], 4, 32)

  y3b = _tc_b(h2pre, st2, g2[None, :], be2[None, :], W3.T, 128, 32, 2)
  agg3b = _segsum_l3(src2, dst2, y3b, z16)
  h3pre, st3 = _tc_c(y3b, agg3b, b3[None, :], 2, 16)

  batchf = batch.astype(jnp.float32).reshape(_N, 1)
  return _tc_f(h3pre, st3, g3[None, :], be3[None, :], lW1.T, lb1[None, :],
               lW2.T, lb2[None, :], batchf)


# speedup vs baseline: 7.5616x; 1.2530x over previous
import functools
import jax
import jax.numpy as jnp
from jax import lax
from jax.experimental import pallas as pl
from jax.experimental.pallas import tpu as pltpu
from jax.experimental.pallas import tpu_sc as plsc

_N = 50000
_E = 800000
_G = 64
_EP = 802816
_ROWS_PT = 3128
_NPAD = 16 * _ROWS_PT
_TR = 1000
_RPT = 392


def _make_segsum(nb, F, rounds, ch, nchunk):
  assert nchunk * ch == _RPT and nchunk % 4 == 0 and nchunk >= 8
  mesh = plsc.VectorSubcoreMesh(core_axis_name="c", subcore_axis_name="s")

  @functools.partial(
      pl.kernel, mesh=mesh,
      compiler_params=pltpu.CompilerParams(use_tc_tiling_on_sc=False),
      out_type=jax.ShapeDtypeStruct((nb, _NPAD, F), jnp.float32),
      scratch_types=[
          (pltpu.VMEM((ch, 2, 128), jnp.int32),) * 4,
          (pltpu.VMEM((ch, 128, F), jnp.float32),) * 2,
          (pltpu.SemaphoreType.DMA,) * 2,
          (pltpu.SemaphoreType.DMA,) * 2,
          (pltpu.SemaphoreType.DMA,) * 4,
          pltpu.VMEM_SHARED((_NPAD, F), jnp.float32),
      ],
  )
  def k(sd_hbm, val_hbm, zrows_hbm, out_hbm, idx_v, rows_v, gsem, ssem, isem,
        acc):
    c = lax.axis_index("c")
    s = lax.axis_index("s")
    row0 = s * _RPT

    for r in range(rounds):
      b = 2 * r + c
      pltpu.sync_copy(zrows_hbm, acc.at[pl.ds(s * _ROWS_PT, _ROWS_PT)])
      plsc.subcore_barrier()

      def idx_load(cc, ii):
        return sd_hbm.at[pl.ds(row0 + cc * ch, ch)], idx_v[ii], isem[ii]

      def gathers(p, ii):
        return [(val_hbm.at[b].at[idx_v[ii].at[j, 0]], rows_v[p].at[j],
                 gsem[p]) for j in range(ch)]

      def scatters(p, ii):
        return [(rows_v[p].at[j], acc.at[idx_v[ii].at[j, 1]], ssem[p])
                for j in range(ch)]

      def step(cc, p, ii, first=False, fire_idx=True, fire_g=True):
        q, i1, i2, i3 = 1 - p, (ii + 1) % 4, (ii + 2) % 4, (ii + 3) % 4
        for a in gathers(p, ii):
          pltpu.make_async_copy(*a).wait()
        for a in scatters(p, ii):
          pltpu.async_copy(*a, add=True)
        if fire_idx:
          pltpu.async_copy(*idx_load(cc + 2, i2))
        if not first:
          for a in scatters(q, i3):
            pltpu.make_async_copy(*a).wait()
        if fire_g:
          pltpu.make_async_copy(*idx_load(cc + 1, i1)).wait()
          for a in gathers(q, i1):
            pltpu.async_copy(*a)

      pltpu.sync_copy(sd_hbm.at[pl.ds(row0, ch)], idx_v[0])
      for a in gathers(0, 0):
        pltpu.async_copy(*a)
      pltpu.async_copy(*idx_load(1, 1))

      step(0, 0, 0, first=True)
      step(1, 1, 1)

      def body(kk, carry):
        cc = 4 * kk + 2
        step(cc, 0, 2)
        step(cc + 1, 1, 3)
        step(cc + 2, 0, 0)
        step(cc + 3, 1, 1)
        return carry

      lax.fori_loop(0, (nchunk - 4) // 4, body, 0)

      step(nchunk - 2, 0, 2, fire_idx=False)
      step(nchunk - 1, 1, 3, fire_idx=False, fire_g=False)
      for a in scatters(1, 3):
        pltpu.make_async_copy(*a).wait()

      plsc.subcore_barrier()
      pltpu.sync_copy(acc.at[pl.ds(s * _ROWS_PT, _ROWS_PT)],
                      out_hbm.at[b].at[pl.ds(s * _ROWS_PT, _ROWS_PT)])
      plsc.subcore_barrier()

  return k


_segsum_l1 = _make_segsum(2, 32, 1, 2, 196)
_segsum_l2 = _make_segsum(4, 32, 2, 2, 196)
_segsum_l3 = _make_segsum(2, 16, 1, 14, 28)


def _unblock(a):
  nb, tr, f = a.shape
  return jnp.transpose(a, (1, 0, 2)).reshape(tr, nb * f)


def _block(a, nb):
  tr, d = a.shape
  return jnp.transpose(a.reshape(tr, nb, d // nb), (1, 0, 2))


def _acc_stats(i, st_ref, y):
  ps = jnp.concatenate([jnp.sum(y, 0, keepdims=True),
                        jnp.sum(y * y, 0, keepdims=True)], axis=0)

  @pl.when(i == 0)
  def _():
    st_ref[...] = jnp.zeros_like(st_ref)

  st_ref[...] += ps


def _bn_from_stats(st, g, be, y):
  m = st[0:1, :] / _N
  var = st[1:2, :] / _N - m * m
  inv = 1.0 / jnp.sqrt(var + 1e-5)
  return (y - m) * inv * g + be


def _tca_body(x_ref, aggb_ref, w_ref, b_ref, y_ref, st_ref):
  i = pl.program_id(0)
  sfeat = x_ref[...] + _unblock(aggb_ref[...])
  y = jnp.dot(sfeat, w_ref[...], preferred_element_type=jnp.float32) + b_ref[...]
  y_ref[...] = y
  _acc_stats(i, st_ref, y)


def _tcb_body(y_ref, st_ref, g_ref, be_ref, w_ref, out_ref, nb):
  h = _bn_from_stats(st_ref[...], g_ref[...], be_ref[...], y_ref[...])
  h = jnp.maximum(h, 0.0)
  y2 = jnp.dot(h, w_ref[...], preferred_element_type=jnp.float32)
  out_ref[...] = _block(y2, nb)


def _tcc_body(yb_ref, aggb_ref, b_ref, h_ref, st_ref):
  i = pl.program_id(0)
  t = _unblock(yb_ref[...]) + _unblock(aggb_ref[...]) + b_ref[...]
  h_ref[...] = t
  _acc_stats(i, st_ref, t)


def _tcf_body(h_ref, st_ref, g_ref, be_ref, w1_ref, c1_ref, w2_ref, c2_ref,
              batch_ref, sums_ref, cnt_ref, mean_ref):
  i = pl.program_id(0)
  h3 = _bn_from_stats(st_ref[...], g_ref[...], be_ref[...], h_ref[...])
  z = jnp.maximum(
      jnp.dot(h3, w1_ref[...], preferred_element_type=jnp.float32)
      + c1_ref[...], 0.0)
  z2 = jnp.dot(z, w2_ref[...], preferred_element_type=jnp.float32) + c2_ref[...]
  p = 1.0 / (1.0 + jnp.exp(-z2))
  gidx = lax.broadcasted_iota(jnp.int32, (1, _G), 1).astype(jnp.float32)
  oh = (batch_ref[...] == gidx).astype(jnp.float32)
  dn = (((0,), (0,)), ((), ()))
  psum = lax.dot_general(oh, p, dn, preferred_element_type=jnp.float32)
  csum = lax.dot_general(oh, jnp.ones_like(p), dn,
                         preferred_element_type=jnp.float32)

  @pl.when(i == 0)
  def _():
    sums_ref[...] = jnp.zeros_like(sums_ref)
    cnt_ref[...] = jnp.zeros_like(cnt_ref)

  sums_ref[...] += psum
  cnt_ref[...] += csum

  @pl.when(i == pl.num_programs(0) - 1)
  def _():
    mean_ref[...] = sums_ref[...] / jnp.maximum(cnt_ref[...], 1.0)


_GRID = _N // _TR


def _rowspec(d):
  return pl.BlockSpec((_TR, d), lambda i: (i, 0))


def _blkspec(nb, f):
  return pl.BlockSpec((nb, _TR, f), lambda i: (0, i, 0))


def _fullspec(shape):
  nd = len(shape)
  return pl.BlockSpec(shape, lambda i, _n=nd: (0,) * _n)


def _tc_a(x, aggb, w1t, b1):
  return pl.pallas_call(
      _tca_body,
      grid=(_GRID,),
      in_specs=[_rowspec(64), _blkspec(2, 32), _fullspec((64, 256)),
                _fullspec((1, 256))],
      out_specs=[_rowspec(256), _fullspec((2, 256))],
      out_shape=[jax.ShapeDtypeStruct((_N, 256), jnp.float32),
                 jax.ShapeDtypeStruct((2, 256), jnp.float32)],
  )(x, aggb, w1t, b1)


def _tc_b(y1, st1, g1, be1, w2t, din, dout, nb):
  body = functools.partial(_tcb_body, nb=nb)
  return pl.pallas_call(
      body,
      grid=(_GRID,),
      in_specs=[_rowspec(din), _fullspec((2, din)), _fullspec((1, din)),
                _fullspec((1, din)), _fullspec((din, dout))],
      out_specs=[_blkspec(nb, dout // nb)],
      out_shape=[jax.ShapeDtypeStruct((nb, _NPAD, dout // nb), jnp.float32)],
  )(y1, st1, g1, be1, w2t)[0]


def _tc_c(yb, aggb, b, nb, f):
  d = nb * f
  return pl.pallas_call(
      _tcc_body,
      grid=(_GRID,),
      in_specs=[_blkspec(nb, f), _blkspec(nb, f), _fullspec((1, d))],
      out_specs=[_rowspec(d), _fullspec((2, d))],
      out_shape=[jax.ShapeDtypeStruct((_N, d), jnp.float32),
                 jax.ShapeDtypeStruct((2, d), jnp.float32)],
  )(yb, aggb, b)


def _tc_f(h3pre, st3, g3, be3, lw1t, lb1, lw2t, lb2, batchf):
  outs = pl.pallas_call(
      _tcf_body,
      grid=(_GRID,),
      in_specs=[_rowspec(32), _fullspec((2, 32)), _fullspec((1, 32)),
                _fullspec((1, 32)), _fullspec((32, 32)), _fullspec((1, 32)),
                _fullspec((32, 1)), _fullspec((1, 1)), _rowspec(1)],
      out_specs=[_fullspec((_G, 1)), _fullspec((_G, 1)), _fullspec((_G, 1))],
      out_shape=[jax.ShapeDtypeStruct((_G, 1), jnp.float32),
                 jax.ShapeDtypeStruct((_G, 1), jnp.float32),
                 jax.ShapeDtypeStruct((_G, 1), jnp.float32)],
  )(h3pre, st3, g3, be3, lw1t, lb1, lw2t, lb2, batchf)
  return outs[2]


def kernel(x, edge_index, batch, node_num, edge_num, start_node, gid,
           checkStatus, W1, b1, g1, be1, W2, b2, g2, be2, W3, b3, g3, be3,
           lW1, lb1, lW2, lb2):
  pad = _EP - _E
  src = jnp.concatenate([edge_index[0], jnp.zeros((pad,), jnp.int32)])
  dst = jnp.concatenate([edge_index[1], jnp.full((pad,), _N, jnp.int32)])
  sd = jnp.stack([src.reshape(_EP // 128, 128),
                  dst.reshape(_EP // 128, 128)], axis=1)
  z32 = jnp.zeros((_ROWS_PT, 32), jnp.float32)
  z16 = jnp.zeros((_ROWS_PT, 16), jnp.float32)

  xb = jnp.transpose(x.reshape(_N, 2, 32), (1, 0, 2))
  agg1b = _segsum_l1(sd, xb, z32)
  y1, st1 = _tc_a(x, agg1b, W1.T, b1[None, :])

  y2b = _tc_b(y1, st1, g1[None, :], be1[None, :], W2.T, 256, 128, 4)
  agg2b = _segsum_l2(sd, y2b, z32)
  h2pre, st2 = _tc_c(y2b, agg2b, b2[None, :], 4, 32)

  y3b = _tc_b(h2pre, st2, g2[None, :], be2[None, :], W3.T, 128, 32, 2)
  agg3b = _segsum_l3(sd, y3b, z16)
  h3pre, st3 = _tc_c(y3b, agg3b, b3[None, :], 2, 16)

  batchf = batch.astype(jnp.float32).reshape(_N, 1)
  return _tc_f(h3pre, st3, g3[None, :], be3[None, :], lW1.T, lb1[None, :],
               lW2.T, lb2[None, :], batchf)
